# baseline (speedup 1.0000x reference)
import functools

import jax
import jax.numpy as jnp
from jax import lax
from jax.experimental import pallas as pl
from jax.experimental.pallas import tpu as pltpu
from jax.experimental.pallas import tpu_sc as plsc

_VOCAB = 1000000
_EMBED = 64
_BATCH = 4096
_SEQ = 200
_B = _BATCH * _SEQ
_NC, _NS = 2, 16
_NW = _NC * _NS
_BPW = _B // _NW
_CHUNK = 128
_NCHUNK = _BPW // _CHUNK

_mesh = plsc.VectorSubcoreMesh(core_axis_name="c", subcore_axis_name="s")


@functools.partial(
    pl.kernel,
    mesh=_mesh,
    compiler_params=pltpu.CompilerParams(use_tc_tiling_on_sc=False),
    out_type=jax.ShapeDtypeStruct((_B, _EMBED), jnp.float32),
    scratch_types=[
        pltpu.VMEM((_NCHUNK, _CHUNK), jnp.int32),
        pltpu.VMEM((_CHUNK, _EMBED), jnp.float32),
        pltpu.SemaphoreType.DMA,
    ],
)
def _gather_sc(idx_hbm, table_hbm, out_hbm, idx_v, rows_v, sem):
    wid = lax.axis_index("s") * _NC + lax.axis_index("c")
    pltpu.sync_copy(idx_hbm.at[pl.ds(wid * _NCHUNK, _NCHUNK)], idx_v)
    base = wid * _BPW

    def body(j, carry):
        pltpu.async_copy(table_hbm.at[idx_v.at[j]], rows_v, sem).wait()
        pltpu.sync_copy(rows_v, out_hbm.at[pl.ds(base + j * _CHUNK, _CHUNK)])
        return carry

    lax.fori_loop(0, _NCHUNK, body, 0)


def kernel(x, table):
    idx = x.reshape(_B // _CHUNK, _CHUNK).astype(jnp.int32)
    out = _gather_sc(idx, table)
    return out.reshape(_BATCH, _SEQ, _EMBED)


# speedup vs baseline: 1.1120x; 1.1120x over previous
import functools

import jax
import jax.numpy as jnp
from jax import lax
from jax.experimental import pallas as pl
from jax.experimental.pallas import tpu as pltpu
from jax.experimental.pallas import tpu_sc as plsc

_VOCAB = 1000000
_EMBED = 64
_BATCH = 4096
_SEQ = 200
_B = _BATCH * _SEQ
_NC, _NS = 2, 16
_NW = _NC * _NS
_BPW = _B // _NW
_CHUNK = 128
_NCHUNK = _BPW // _CHUNK
_K = 2
_SUP = _K * _CHUNK
_NSUP = _BPW // _SUP
_NBUF = 4
_T = _NSUP // _NBUF

_mesh = plsc.VectorSubcoreMesh(core_axis_name="c", subcore_axis_name="s")


@functools.partial(
    pl.kernel,
    mesh=_mesh,
    compiler_params=pltpu.CompilerParams(use_tc_tiling_on_sc=False),
    out_type=jax.ShapeDtypeStruct((_B, _EMBED), jnp.float32),
    scratch_types=[
        pltpu.VMEM((_NCHUNK, _CHUNK), jnp.int32),
        pltpu.VMEM((_NBUF, _SUP, _EMBED), jnp.float32),
    ] + [pltpu.SemaphoreType.DMA] * (2 * _NBUF),
)
def _gather_sc(idx_hbm, table_hbm, out_hbm, idx_v, rows_v, *sems):
    gsem = sems[:_NBUF]
    ssem = sems[_NBUF:]
    wid = lax.axis_index("s") * _NC + lax.axis_index("c")
    pltpu.sync_copy(idx_hbm.at[pl.ds(wid * _NCHUNK, _NCHUNK)], idx_v)
    base = wid * _BPW

    def issue_gather(sup, b):
        for k in range(_K):
            pltpu.async_copy(
                table_hbm.at[idx_v.at[sup * _K + k]],
                rows_v.at[b, pl.ds(k * _CHUNK, _CHUNK)],
                gsem[b],
            )

    def wait_gather(b):
        for k in range(_K):
            pltpu.make_async_copy(
                table_hbm.at[idx_v.at[0]],
                rows_v.at[b, pl.ds(k * _CHUNK, _CHUNK)],
                gsem[b],
            ).wait()

    def issue_scatter(sup, b):
        pltpu.async_copy(
            rows_v.at[b], out_hbm.at[pl.ds(base + sup * _SUP, _SUP)], ssem[b]
        )

    def wait_scatter(b):
        pltpu.make_async_copy(
            rows_v.at[b], out_hbm.at[pl.ds(base, _SUP)], ssem[b]
        ).wait()

    def step(sup, b, do_issue, do_wait_prev):
        pb = (b - 1) % _NBUF
        wait_gather(b)
        issue_scatter(sup, b)
        if do_wait_prev:
            wait_scatter(pb)
        if do_issue:
            issue_gather(sup + _NBUF - 1, pb)

    for b in range(_NBUF - 1):
        issue_gather(b, b)
    step(0, 0, True, False)
    for b in range(1, _NBUF):
        step(b, b, True, True)

    def body(t, carry):
        for b in range(_NBUF):
            step(t * _NBUF + b, b, True, True)
        return carry

    lax.fori_loop(1, _T - 1, body, 0)

    s0 = (_T - 1) * _NBUF
    step(s0, 0, True, True)
    for b in range(1, _NBUF):
        step(s0 + b, b, False, True)
    wait_scatter(_NBUF - 1)


def kernel(x, table):
    idx = x.reshape(_B // _CHUNK, _CHUNK).astype(jnp.int32)
    out = _gather_sc(idx, table)
    return out.reshape(_BATCH, _SEQ, _EMBED)
